# baseline (speedup 1.0000x reference)
import functools

import jax
import jax.numpy as jnp
from jax import lax
from jax.experimental import pallas as pl
from jax.experimental.pallas import tpu as pltpu
from jax.experimental.pallas import tpu_sc as plsc

B, S, D = 4, 8192, 768
E = 64
K = 8
SEU = B * S
L = 16
P = 4

_S8 = [(0, 1), (2, 3), (4, 5), (6, 7),
       (0, 2), (1, 3), (4, 6), (5, 7),
       (1, 2), (5, 6),
       (0, 4), (1, 5), (2, 6), (3, 7),
       (2, 4), (3, 5),
       (1, 2), (3, 4), (5, 6)]


def _sort8_desc(v):
    v = list(v)
    for i, j in _S8:
        a, b = v[i], v[j]
        v[i] = jnp.maximum(a, b)
        v[j] = jnp.minimum(a, b)
    return v


def _merge_top8(a, b):
    t = [jnp.maximum(a[i], b[7 - i]) for i in range(8)]
    for d in (4, 2, 1):
        for i in range(8):
            if (i % (2 * d)) < d:
                hi = jnp.maximum(t[i], t[i + d])
                lo = jnp.minimum(t[i], t[i + d])
                t[i], t[i + d] = hi, lo
    return t


_TC_BLK = 4096


_STREAMS = 2


def _make_matmul_body(rpw, cpw):
    def _matmul_body(*refs):
        h_refs, w_ref, out_ref = refs[:_STREAMS], refs[_STREAMS], refs[-1]
        for s, h_ref in enumerate(h_refs):
            res = lax.dot_general(
                w_ref[...], h_ref[...],
                dimension_numbers=(((1,), (1,)), ((), ())),
                preferred_element_type=jnp.float32)
            for c in range(_TC_BLK // rpw):
                for p in range(P):
                    base = c * rpw + p * cpw
                    out_ref[s * (_TC_BLK // rpw) + c, p] = \
                        res[:, base:base + cpw]
    return _matmul_body


def _logits_blocked(hflat, w, nw):
    rpw = SEU // nw
    cpw = rpw // P
    grid = SEU // (_TC_BLK * _STREAMS)
    sub = _TC_BLK // rpw

    def h_spec(s):
        return pl.BlockSpec((_TC_BLK, D), lambda i, s=s: (_STREAMS * i + s, 0))

    return pl.pallas_call(
        _make_matmul_body(rpw, cpw),
        grid=(grid,),
        in_specs=[h_spec(s) for s in range(_STREAMS)] + [
            pl.BlockSpec((E, D), lambda i: (0, 0)),
        ],
        out_specs=pl.BlockSpec(
            (_STREAMS * sub, P, E, cpw), lambda i: (i, 0, 0, 0)),
        out_shape=jax.ShapeDtypeStruct((nw, P, E, cpw), jnp.float32),
    )(*([hflat] * _STREAMS), w)


@functools.cache
def _make_sc_topk():
    info = plsc.get_sparse_core_info()
    nc, ns = info.num_cores, info.num_subcores
    nw = nc * ns
    rpw = SEU // nw
    cpw = rpw // P
    cgroups = cpw // L
    blocks = E // K

    mesh = plsc.VectorSubcoreMesh(core_axis_name="c", subcore_axis_name="s")

    @functools.partial(
        pl.kernel,
        out_type=jax.ShapeDtypeStruct((nw, K, rpw), jnp.float32),
        mesh=mesh,
        scratch_types=[
            pltpu.VMEM((2, E, cpw), jnp.float32),
            pltpu.VMEM((K, rpw), jnp.float32),
            pltpu.SemaphoreType.DMA,
            pltpu.SemaphoreType.DMA,
        ],
    )
    def sc_topk(logits_hbm, out_hbm, lbuf, oblk, sem0, sem1):
        wid = lax.axis_index("s") * nc + lax.axis_index("c")
        sems = (sem0, sem1)
        copies = [None] * P
        copies[0] = pltpu.async_copy(logits_hbm.at[wid, 0], lbuf.at[0], sem0)
        for p in range(P):
            if p + 1 < P:
                copies[p + 1] = pltpu.async_copy(
                    logits_hbm.at[wid, p + 1], lbuf.at[(p + 1) % 2],
                    sems[(p + 1) % 2])
            copies[p].wait()
            buf = p % 2

            def group_body(g, carry, buf=buf, p=p):
                col = g * L
                acc = _sort8_desc(
                    [lbuf[buf, e, pl.ds(col, L)] for e in range(K)])
                for blk in range(1, blocks):
                    cand = _sort8_desc(
                        [lbuf[buf, blk * K + t, pl.ds(col, L)]
                         for t in range(K)])
                    acc = _merge_top8(acc, cand)
                exps = [jnp.exp(a - acc[0]) for a in acc]
                ssum = exps[0]
                for j in range(1, K):
                    ssum = ssum + exps[j]
                inv = jnp.float32(1.0) / ssum
                for j in range(K):
                    oblk[j, pl.ds(p * cpw + col, L)] = exps[j] * inv
                return carry

            lax.fori_loop(0, cgroups, group_body, 0)
        pltpu.sync_copy(oblk, out_hbm.at[wid])

    return sc_topk, nw


def kernel(hidden_states, kernel):
    sc_topk, nw = _make_sc_topk()
    hflat = hidden_states.reshape(SEU, D)
    logits = _logits_blocked(hflat, kernel, nw)
    out_blk = sc_topk(logits)
    return out_blk.transpose(0, 2, 1).reshape(SEU, K)


# speedup vs baseline: 1.0333x; 1.0333x over previous
import functools

import jax
import jax.numpy as jnp
from jax import lax
from jax.experimental import pallas as pl
from jax.experimental.pallas import tpu as pltpu
from jax.experimental.pallas import tpu_sc as plsc

B, S, D = 4, 8192, 768
E = 64
K = 8
SEU = B * S
L = 16
P = 4

_S8 = [(0, 1), (2, 3), (4, 5), (6, 7),
       (0, 2), (1, 3), (4, 6), (5, 7),
       (1, 2), (5, 6),
       (0, 4), (1, 5), (2, 6), (3, 7),
       (2, 4), (3, 5),
       (1, 2), (3, 4), (5, 6)]


def _sort8_desc(v):
    v = list(v)
    for i, j in _S8:
        a, b = v[i], v[j]
        v[i] = jnp.maximum(a, b)
        v[j] = jnp.minimum(a, b)
    return v


def _merge_top8(a, b):
    t = [jnp.maximum(a[i], b[7 - i]) for i in range(8)]
    for d in (4, 2, 1):
        for i in range(8):
            if (i % (2 * d)) < d:
                hi = jnp.maximum(t[i], t[i + d])
                lo = jnp.minimum(t[i], t[i + d])
                t[i], t[i + d] = hi, lo
    return t


_TC_BLK = 2048


_STREAMS = 2


def _make_matmul_body(rpw, cpw):
    def _matmul_body(*refs):
        h_refs, w_ref, out_ref = refs[:_STREAMS], refs[_STREAMS], refs[-1]
        for s, h_ref in enumerate(h_refs):
            res = lax.dot_general(
                w_ref[...], h_ref[...],
                dimension_numbers=(((1,), (1,)), ((), ())),
                preferred_element_type=jnp.float32)
            for c in range(_TC_BLK // rpw):
                for p in range(P):
                    base = c * rpw + p * cpw
                    out_ref[s * (_TC_BLK // rpw) + c, p] = \
                        res[:, base:base + cpw]
    return _matmul_body


def _logits_blocked(hflat, w, nw):
    rpw = SEU // nw
    cpw = rpw // P
    grid = SEU // (_TC_BLK * _STREAMS)
    sub = _TC_BLK // rpw

    def h_spec(s):
        return pl.BlockSpec((_TC_BLK, D), lambda i, s=s: (_STREAMS * i + s, 0))

    return pl.pallas_call(
        _make_matmul_body(rpw, cpw),
        grid=(grid,),
        in_specs=[h_spec(s) for s in range(_STREAMS)] + [
            pl.BlockSpec((E, D), lambda i: (0, 0)),
        ],
        out_specs=pl.BlockSpec(
            (_STREAMS * sub, P, E, cpw), lambda i: (i, 0, 0, 0)),
        out_shape=jax.ShapeDtypeStruct((nw, P, E, cpw), jnp.float32),
    )(*([hflat] * _STREAMS), w)


@functools.cache
def _make_sc_topk():
    info = plsc.get_sparse_core_info()
    nc, ns = info.num_cores, info.num_subcores
    nw = nc * ns
    rpw = SEU // nw
    cpw = rpw // P
    cgroups = cpw // L
    blocks = E // K

    mesh = plsc.VectorSubcoreMesh(core_axis_name="c", subcore_axis_name="s")

    @functools.partial(
        pl.kernel,
        out_type=jax.ShapeDtypeStruct((nw, K, rpw), jnp.float32),
        mesh=mesh,
        scratch_types=[
            pltpu.VMEM((2, E, cpw), jnp.float32),
            pltpu.VMEM((K, rpw), jnp.float32),
            pltpu.SemaphoreType.DMA,
            pltpu.SemaphoreType.DMA,
        ],
    )
    def sc_topk(logits_hbm, out_hbm, lbuf, oblk, sem0, sem1):
        wid = lax.axis_index("s") * nc + lax.axis_index("c")
        sems = (sem0, sem1)
        copies = [None] * P
        copies[0] = pltpu.async_copy(logits_hbm.at[wid, 0], lbuf.at[0], sem0)
        for p in range(P):
            if p + 1 < P:
                copies[p + 1] = pltpu.async_copy(
                    logits_hbm.at[wid, p + 1], lbuf.at[(p + 1) % 2],
                    sems[(p + 1) % 2])
            copies[p].wait()
            buf = p % 2

            def group_body(g, carry, buf=buf, p=p):
                col = g * L
                acc = _sort8_desc(
                    [lbuf[buf, e, pl.ds(col, L)] for e in range(K)])
                for blk in range(1, blocks):
                    cand = _sort8_desc(
                        [lbuf[buf, blk * K + t, pl.ds(col, L)]
                         for t in range(K)])
                    acc = _merge_top8(acc, cand)
                exps = [jnp.exp(a - acc[0]) for a in acc]
                ssum = exps[0]
                for j in range(1, K):
                    ssum = ssum + exps[j]
                inv = jnp.float32(1.0) / ssum
                for j in range(K):
                    oblk[j, pl.ds(p * cpw + col, L)] = exps[j] * inv
                return carry

            lax.fori_loop(0, cgroups, group_body, 0)
        pltpu.sync_copy(oblk, out_hbm.at[wid])

    return sc_topk, nw


def kernel(hidden_states, kernel):
    sc_topk, nw = _make_sc_topk()
    hflat = hidden_states.reshape(SEU, D)
    logits = _logits_blocked(hflat, kernel, nw)
    out_blk = sc_topk(logits)
    return out_blk.transpose(0, 2, 1).reshape(SEU, K)


# speedup vs baseline: 1.0341x; 1.0007x over previous
import functools

import jax
import jax.numpy as jnp
from jax import lax
from jax.experimental import pallas as pl
from jax.experimental.pallas import tpu as pltpu
from jax.experimental.pallas import tpu_sc as plsc

B, S, D = 4, 8192, 768
E = 64
K = 8
SEU = B * S
L = 16
P = 4

_S8 = [(0, 1), (2, 3), (4, 5), (6, 7),
       (0, 2), (1, 3), (4, 6), (5, 7),
       (1, 2), (5, 6),
       (0, 4), (1, 5), (2, 6), (3, 7),
       (2, 4), (3, 5),
       (1, 2), (3, 4), (5, 6)]


def _sort8_desc(v):
    v = list(v)
    for i, j in _S8:
        a, b = v[i], v[j]
        v[i] = jnp.maximum(a, b)
        v[j] = jnp.minimum(a, b)
    return v


def _merge_top8(a, b):
    t = [jnp.maximum(a[i], b[7 - i]) for i in range(8)]
    for d in (4, 2, 1):
        for i in range(8):
            if (i % (2 * d)) < d:
                hi = jnp.maximum(t[i], t[i + d])
                lo = jnp.minimum(t[i], t[i + d])
                t[i], t[i + d] = hi, lo
    return t


_TC_BLK = 4096


_STREAMS = 1


def _make_matmul_body(rpw, cpw):
    def _matmul_body(*refs):
        h_refs, w_ref, out_ref = refs[:_STREAMS], refs[_STREAMS], refs[-1]
        for s, h_ref in enumerate(h_refs):
            res = lax.dot_general(
                w_ref[...], h_ref[...],
                dimension_numbers=(((1,), (1,)), ((), ())),
                preferred_element_type=jnp.float32)
            for c in range(_TC_BLK // rpw):
                for p in range(P):
                    base = c * rpw + p * cpw
                    out_ref[s * (_TC_BLK // rpw) + c, p] = \
                        res[:, base:base + cpw]
    return _matmul_body


def _logits_blocked(hflat, w, nw):
    rpw = SEU // nw
    cpw = rpw // P
    grid = SEU // (_TC_BLK * _STREAMS)
    sub = _TC_BLK // rpw

    def h_spec(s):
        return pl.BlockSpec((_TC_BLK, D), lambda i, s=s: (_STREAMS * i + s, 0))

    return pl.pallas_call(
        _make_matmul_body(rpw, cpw),
        grid=(grid,),
        in_specs=[h_spec(s) for s in range(_STREAMS)] + [
            pl.BlockSpec((E, D), lambda i: (0, 0)),
        ],
        out_specs=pl.BlockSpec(
            (_STREAMS * sub, P, E, cpw), lambda i: (i, 0, 0, 0)),
        out_shape=jax.ShapeDtypeStruct((nw, P, E, cpw), jnp.float32),
    )(*([hflat] * _STREAMS), w)


@functools.cache
def _make_sc_topk():
    info = plsc.get_sparse_core_info()
    nc, ns = info.num_cores, info.num_subcores
    nw = nc * ns
    rpw = SEU // nw
    cpw = rpw // P
    cgroups = cpw // L
    blocks = E // K

    mesh = plsc.VectorSubcoreMesh(core_axis_name="c", subcore_axis_name="s")

    @functools.partial(
        pl.kernel,
        out_type=jax.ShapeDtypeStruct((nw, K, rpw), jnp.float32),
        mesh=mesh,
        scratch_types=[
            pltpu.VMEM((2, E, cpw), jnp.float32),
            pltpu.VMEM((K, rpw), jnp.float32),
            pltpu.SemaphoreType.DMA,
            pltpu.SemaphoreType.DMA,
        ],
    )
    def sc_topk(logits_hbm, out_hbm, lbuf, oblk, sem0, sem1):
        wid = lax.axis_index("s") * nc + lax.axis_index("c")
        sems = (sem0, sem1)
        copies = [None] * P
        copies[0] = pltpu.async_copy(logits_hbm.at[wid, 0], lbuf.at[0], sem0)
        for p in range(P):
            if p + 1 < P:
                copies[p + 1] = pltpu.async_copy(
                    logits_hbm.at[wid, p + 1], lbuf.at[(p + 1) % 2],
                    sems[(p + 1) % 2])
            copies[p].wait()
            buf = p % 2

            def group_body(g, carry, buf=buf, p=p):
                col = g * L
                acc = _sort8_desc(
                    [lbuf[buf, e, pl.ds(col, L)] for e in range(K)])
                for blk in range(1, blocks):
                    cand = _sort8_desc(
                        [lbuf[buf, blk * K + t, pl.ds(col, L)]
                         for t in range(K)])
                    acc = _merge_top8(acc, cand)
                exps = [jnp.exp(a - acc[0]) for a in acc]
                ssum = exps[0]
                for j in range(1, K):
                    ssum = ssum + exps[j]
                inv = jnp.float32(1.0) / ssum
                for j in range(K):
                    oblk[j, pl.ds(p * cpw + col, L)] = exps[j] * inv
                return carry

            lax.fori_loop(0, cgroups, group_body, 0)
        pltpu.sync_copy(oblk, out_hbm.at[wid])

    return sc_topk, nw


def kernel(hidden_states, kernel):
    sc_topk, nw = _make_sc_topk()
    hflat = hidden_states.reshape(SEU, D)
    logits = _logits_blocked(hflat, kernel, nw)
    out_blk = sc_topk(logits)
    return out_blk.transpose(0, 2, 1).reshape(SEU, K)
